# baseline (speedup 1.0000x reference)
import jax
import jax.numpy as jnp
from jax import lax
from jax.experimental import pallas as pl
from jax.experimental.pallas import tpu as pltpu
from jax.experimental.pallas import tpu_sc as plsc

BATCH = 16384
DIM = 128
NC = 2
NS = 16
NW = NC * NS
BPW = BATCH // NW
CH = 64
NCHUNK = BPW // CH
LANES = 16
NBLK = DIM // (2 * LANES)
GROUPS = CH // LANES
NROW = 4
SPAD = LANES + 1


def _complex_score_body(idx_ht_hbm, idx_r_hbm, ent_re, ent_im, rel_cat,
                        out_hbm, idx_ht, idx_r, bre, bim, brel,
                        stage, out_v, sem0, sem1, sem2):
    wid = lax.axis_index("s") * NC + lax.axis_index("c")

    pltpu.sync_copy(idx_ht_hbm.at[pl.ds(wid * 2 * BPW, 2 * BPW)], idx_ht)
    pltpu.sync_copy(idx_r_hbm.at[pl.ds(wid * BPW, BPW)], idx_r)

    def copies(g, parity):
        sem = (sem0, sem1, sem2)[parity]
        sl_ht = pl.ds(g * 2 * CH, 2 * CH)
        sl_r = pl.ds(g * CH, CH)
        return [
            (ent_re.at[idx_ht.at[sl_ht]], bre.at[parity], sem),
            (ent_im.at[idx_ht.at[sl_ht]], bim.at[parity], sem),
            (rel_cat.at[idx_r.at[sl_r]], brel.at[parity], sem),
        ]

    def issue(g, parity):
        for src, dst, sem in copies(g, parity):
            pltpu.async_copy(src, dst, sem)

    def drain(g, parity):
        for src, dst, sem in copies(g, parity):
            pltpu.make_async_copy(src, dst, sem).wait()

    lane_iota = lax.iota(jnp.int32, LANES)

    def compute(g, p):
        def group_body(gi, carry):
            row0 = gi * LANES

            def jblk_body(jb, carry2):
                j = jb * NROW
                ii = tuple(row0 + j + k for k in range(NROW))
                acc = [[jnp.zeros((LANES,), jnp.float32) for _ in range(2)]
                       for _ in range(NROW)]
                for blk in range(NBLK):
                    rels = []
                    for row in range(NROW):
                        vrr = plsc.bitcast(
                            brel[p, ii[row], pl.ds(blk * LANES, LANES)],
                            jnp.bfloat16)
                        vri = plsc.bitcast(
                            brel[p, ii[row],
                                 pl.ds(DIM // 2 + blk * LANES, LANES)],
                            jnp.bfloat16)
                        rr0, rr1 = plsc.unpack(
                            vrr, format=plsc.PackFormat.INTERLEAVED)
                        ri0, ri1 = plsc.unpack(
                            vri, format=plsc.PackFormat.INTERLEAVED)
                        rels.append(((rr0, ri0), (rr1, ri1)))
                    for half in range(2):
                        dsl = pl.ds((2 * blk + half) * LANES, LANES)
                        for row in range(NROW):
                            e, f = rels[row][half]
                            i = ii[row]
                            a = bre[p, i, dsl]
                            b = bim[p, i, dsl]
                            cc = bre[p, CH + i, dsl]
                            dd = bim[p, CH + i, dsl]
                            acc[row][0] = acc[row][0] + e * (a * cc + b * dd)
                            acc[row][1] = acc[row][1] + f * (a * dd - b * cc)
                for row in range(NROW - 1, -1, -1):
                    stage[j + row, pl.ds(0, LANES)] = (acc[row][0]
                                                       + acc[row][1])
                return carry2

            lax.fori_loop(0, LANES // NROW, jblk_body, 0)
            cols = [plsc.load_gather(stage,
                                     [lane_iota, jnp.full((LANES,), c,
                                                          jnp.int32)])
                    for c in range(LANES)]
            while len(cols) > 1:
                cols = [cols[k] + cols[k + 1] for k in range(0, len(cols), 2)]
            out_v[pl.ds(g * CH + row0, LANES)] = cols[0]
            return carry

        lax.fori_loop(0, GROUPS, group_body, 0)

    issue(0, 0)
    issue(1, 1)

    def chunk_body(g, carry):
        p = lax.rem(g, 3)
        pn = lax.rem(g + 2, 3)

        for k in range(3):
            @pl.when(jnp.logical_and(g + 2 < NCHUNK, pn == k))
            def _(k=k):
                issue(g + 2, k)

        for k in range(3):
            @pl.when(p == k)
            def _(k=k):
                drain(g, k)

        compute(g, p)
        return carry

    lax.fori_loop(0, NCHUNK, chunk_body, 0)

    pltpu.sync_copy(out_v, out_hbm.at[pl.ds(wid * BPW, BPW)])


@jax.jit
def _complex_score(h, r, t, ent_re, ent_im, rel_re, rel_im):
    hh = h.reshape(NW, NCHUNK, 1, CH)
    tt = t.reshape(NW, NCHUNK, 1, CH)
    idx_ht = jnp.concatenate([hh, tt], axis=2).reshape(-1)

    def perm(x):
        return x.reshape(-1, NBLK, 2, LANES).transpose(0, 1, 3, 2).reshape(
            -1, DIM)

    rel_cat16 = jnp.concatenate(
        [perm(rel_re), perm(rel_im)], axis=1).astype(jnp.bfloat16)
    rel_cat = jax.lax.bitcast_convert_type(
        rel_cat16.reshape(-1, DIM, 2), jnp.int32)

    mesh = plsc.VectorSubcoreMesh(core_axis_name="c", subcore_axis_name="s")
    kfn = pl.kernel(
        _complex_score_body,
        out_type=jax.ShapeDtypeStruct((BATCH,), jnp.float32),
        mesh=mesh,
        compiler_params=pltpu.CompilerParams(needs_layout_passes=False),
        scratch_types=[
            pltpu.VMEM((2 * BPW,), jnp.int32),
            pltpu.VMEM((BPW,), jnp.int32),
            pltpu.VMEM((3, 2 * CH, DIM), jnp.float32),
            pltpu.VMEM((3, 2 * CH, DIM), jnp.float32),
            pltpu.VMEM((3, CH, DIM), jnp.int32),
            pltpu.VMEM((LANES, SPAD), jnp.float32),
            pltpu.VMEM((BPW,), jnp.float32),
            pltpu.SemaphoreType.DMA,
            pltpu.SemaphoreType.DMA,
            pltpu.SemaphoreType.DMA,
        ],
    )
    return kfn(idx_ht, r, ent_re, ent_im, rel_cat)


def kernel(h, r, t, ent_re, ent_im, rel_re, rel_im):
    return _complex_score(h.astype(jnp.int32), r.astype(jnp.int32),
                          t.astype(jnp.int32), ent_re, ent_im, rel_re, rel_im)


# speedup vs baseline: 1.0224x; 1.0224x over previous
import jax
import jax.numpy as jnp
from jax import lax
from jax.experimental import pallas as pl
from jax.experimental.pallas import tpu as pltpu
from jax.experimental.pallas import tpu_sc as plsc

BATCH = 16384
DIM = 128
NC = 2
NS = 16
NW = NC * NS
BPW = BATCH // NW
CH = 64
NCHUNK = BPW // CH
LANES = 16
NBLK = DIM // (2 * LANES)
GROUPS = CH // LANES
NROW = 4
SPAD = LANES + 1


def _complex_score_body(idx_ht_hbm, idx_r_hbm, ent_re, ent_im, rel_cat,
                        out_hbm, idx_ht, idx_r, bre, bim, brel,
                        stage, out_v, sem0, sem1):
    wid = lax.axis_index("s") * NC + lax.axis_index("c")

    cp1 = pltpu.async_copy(idx_ht_hbm.at[pl.ds(wid * 2 * BPW, 2 * BPW)],
                           idx_ht, sem0)
    cp2 = pltpu.async_copy(idx_r_hbm.at[pl.ds(wid * BPW, BPW)], idx_r, sem1)
    cp1.wait()
    cp2.wait()

    def copies(g, parity):
        sem = sem0 if parity == 0 else sem1
        sl_ht = pl.ds(g * 2 * CH, 2 * CH)
        sl_r = pl.ds(g * CH, CH)
        return [
            (ent_re.at[idx_ht.at[sl_ht]], bre.at[parity], sem),
            (ent_im.at[idx_ht.at[sl_ht]], bim.at[parity], sem),
            (rel_cat.at[idx_r.at[sl_r]], brel.at[parity], sem),
        ]

    def issue(g, parity):
        for src, dst, sem in copies(g, parity):
            pltpu.async_copy(src, dst, sem)

    def drain(g, parity):
        for src, dst, sem in copies(g, parity):
            pltpu.make_async_copy(src, dst, sem).wait()

    lane_iota = lax.iota(jnp.int32, LANES)

    def compute(g, p):
        def group_body(gi, carry):
            row0 = gi * LANES

            def jblk_body(jb, carry2):
                j = jb * NROW
                ii = tuple(row0 + j + k for k in range(NROW))
                acc = [[jnp.zeros((LANES,), jnp.float32) for _ in range(2)]
                       for _ in range(NROW)]
                for blk in range(NBLK):
                    rels = []
                    for row in range(NROW):
                        vrr = plsc.bitcast(
                            brel[p, ii[row], pl.ds(blk * LANES, LANES)],
                            jnp.bfloat16)
                        vri = plsc.bitcast(
                            brel[p, ii[row],
                                 pl.ds(DIM // 2 + blk * LANES, LANES)],
                            jnp.bfloat16)
                        rr0, rr1 = plsc.unpack(
                            vrr, format=plsc.PackFormat.INTERLEAVED)
                        ri0, ri1 = plsc.unpack(
                            vri, format=plsc.PackFormat.INTERLEAVED)
                        rels.append(((rr0, ri0), (rr1, ri1)))
                    for half in range(2):
                        dsl = pl.ds((2 * blk + half) * LANES, LANES)
                        for row in range(NROW):
                            e, f = rels[row][half]
                            i = ii[row]
                            a = bre[p, i, dsl]
                            b = bim[p, i, dsl]
                            cc = bre[p, CH + i, dsl]
                            dd = bim[p, CH + i, dsl]
                            acc[row][0] = acc[row][0] + e * (a * cc + b * dd)
                            acc[row][1] = acc[row][1] + f * (a * dd - b * cc)
                for row in range(NROW - 1, -1, -1):
                    stage[j + row, pl.ds(0, LANES)] = (acc[row][0]
                                                       + acc[row][1])
                return carry2

            lax.fori_loop(0, LANES // NROW, jblk_body, 0)
            cols = [plsc.load_gather(stage,
                                     [lane_iota, jnp.full((LANES,), c,
                                                          jnp.int32)])
                    for c in range(LANES)]
            while len(cols) > 1:
                cols = [cols[k] + cols[k + 1] for k in range(0, len(cols), 2)]
            out_v[pl.ds(g * CH + row0, LANES)] = cols[0]
            return carry

        lax.fori_loop(0, GROUPS, group_body, 0)

    issue(0, 0)

    def chunk_body(g, carry):
        p = jnp.bitwise_and(g, 1)

        @pl.when(jnp.logical_and(g + 1 < NCHUNK, p == 0))
        def _():
            issue(g + 1, 1)

        @pl.when(jnp.logical_and(g + 1 < NCHUNK, p == 1))
        def _():
            issue(g + 1, 0)

        @pl.when(p == 0)
        def _():
            drain(g, 0)

        @pl.when(p == 1)
        def _():
            drain(g, 1)

        compute(g, p)
        return carry

    lax.fori_loop(0, NCHUNK, chunk_body, 0)

    pltpu.sync_copy(out_v, out_hbm.at[pl.ds(wid * BPW, BPW)])


@jax.jit
def _complex_score(h, r, t, ent_re, ent_im, rel_re, rel_im):
    hh = h.reshape(NW, NCHUNK, 1, CH)
    tt = t.reshape(NW, NCHUNK, 1, CH)
    idx_ht = jnp.concatenate([hh, tt], axis=2).reshape(-1)

    def perm(x):
        return x.reshape(-1, NBLK, 2, LANES).transpose(0, 1, 3, 2).reshape(
            -1, DIM)

    rel_cat16 = jnp.concatenate(
        [perm(rel_re), perm(rel_im)], axis=1).astype(jnp.bfloat16)
    rel_cat = jax.lax.bitcast_convert_type(
        rel_cat16.reshape(-1, DIM, 2), jnp.int32)

    mesh = plsc.VectorSubcoreMesh(core_axis_name="c", subcore_axis_name="s")
    kfn = pl.kernel(
        _complex_score_body,
        out_type=jax.ShapeDtypeStruct((BATCH,), jnp.float32),
        mesh=mesh,
        compiler_params=pltpu.CompilerParams(needs_layout_passes=False),
        scratch_types=[
            pltpu.VMEM((2 * BPW,), jnp.int32),
            pltpu.VMEM((BPW,), jnp.int32),
            pltpu.VMEM((2, 2 * CH, DIM), jnp.float32),
            pltpu.VMEM((2, 2 * CH, DIM), jnp.float32),
            pltpu.VMEM((2, CH, DIM), jnp.int32),
            pltpu.VMEM((LANES, SPAD), jnp.float32),
            pltpu.VMEM((BPW,), jnp.float32),
            pltpu.SemaphoreType.DMA,
            pltpu.SemaphoreType.DMA,
        ],
    )
    return kfn(idx_ht, r, ent_re, ent_im, rel_cat)


def kernel(h, r, t, ent_re, ent_im, rel_re, rel_im):
    return _complex_score(h.astype(jnp.int32), r.astype(jnp.int32),
                          t.astype(jnp.int32), ent_re, ent_im, rel_re, rel_im)
